# baseline (speedup 1.0000x reference)
import functools

import jax
import jax.numpy as jnp
from jax import lax
from jax.experimental import pallas as pl
from jax.experimental.pallas import tpu as pltpu
from jax.experimental.pallas import tpu_sc as plsc

F32 = jnp.float32
NC, NS = 2, 16
BLK = 128
BM = 256
HID = 256
NSLAB = 2
SLAB = HID // NSLAB
MAXACC = 12800
DUMMY = 256

_NODE = ['query', 'textblock', 'table', 'cell', 'image', 'caption']
_EDGES = [
    ('query', 'query_to_text', 'textblock'),
    ('query', 'query_to_cell', 'cell'),
    ('textblock', 'text_to_query', 'query'),
    ('cell', 'cell_to_query', 'query'),
    ('table', 'table_contains_cell', 'cell'),
    ('cell', 'cell_in_table', 'table'),
    ('cell', 'cell_to_cell_row', 'cell'),
    ('cell', 'cell_to_cell_col', 'cell'),
    ('textblock', 'text_refers_table', 'table'),
    ('image', 'image_has_caption', 'caption'),
    ('caption', 'caption_to_image', 'image'),
]
_SCORED = ['textblock', 'cell', 'image', 'caption', 'table']
_RELS_OF = {t: [i for i, e in enumerate(_EDGES) if e[2] == t] for t in _NODE}
_CNT_SPLIT = 6


def _rup(x, m):
    return -(-x // m) * m


def _relu(x):
    return jnp.maximum(x, 0.0)


def _proj_body(x_ref, w_ref, b_ref, o_ref):
    h = _relu(jnp.dot(x_ref[...], w_ref[...], preferred_element_type=F32)
              + b_ref[...])
    for k in range(NSLAB):
        o_ref[k] = h[:, SLAB * k:SLAB * (k + 1)]


def _proj(x_pad, w, b, npad):
    return pl.pallas_call(
        _proj_body,
        grid=(npad // BM,),
        in_specs=[
            pl.BlockSpec((BM, x_pad.shape[1]), lambda m: (m, 0)),
            pl.BlockSpec(w.shape, lambda m: (0, 0)),
            pl.BlockSpec((1, HID), lambda m: (0, 0)),
        ],
        out_specs=pl.BlockSpec((NSLAB, BM, SLAB), lambda m: (0, m, 0)),
        out_shape=jax.ShapeDtypeStruct((NSLAB, npad, SLAB), F32),
    )(x_pad, w, b.reshape(1, HID))


def _layer_body(nrel, x_ref, w_ref, *rest):
    s_refs = rest[:nrel]
    c_refs = rest[nrel:2 * nrel]
    o_ref = rest[2 * nrel]
    h = jnp.concatenate([x_ref[k] for k in range(NSLAB)], axis=1)
    acc = jnp.dot(h, w_ref[0:HID], preferred_element_type=F32)
    for j in range(nrel):
        r = 1.0 / jnp.maximum(c_refs[j][:, 0:1], 1.0)
        sj = jnp.concatenate([s_refs[j][k] for k in range(NSLAB)], axis=1)
        acc = acc + jnp.dot(sj * r,
                            w_ref[HID * (j + 1):HID * (j + 2)],
                            preferred_element_type=F32)
    out = _relu(acc)
    for k in range(NSLAB):
        o_ref[k] = out[:, SLAB * k:SLAB * (k + 1)]


def _layer_mm(x, s_hbm, cnt2, wstk, npad, soffs):
    nrel = len(soffs)
    in_specs = [
        pl.BlockSpec((NSLAB, BM, SLAB), lambda m: (0, m, 0)),
        pl.BlockSpec(wstk.shape, lambda m: (0, 0)),
    ]
    for soff in soffs:
        in_specs.append(
            pl.BlockSpec((NSLAB, BM, SLAB),
                         functools.partial(lambda so, m: (0, m + so, 0),
                                           soff // BM)))
    for soff in soffs:
        in_specs.append(
            pl.BlockSpec((BM, 16),
                         functools.partial(lambda so, m: (m + so, 0),
                                           soff // BM)))
    return pl.pallas_call(
        functools.partial(_layer_body, nrel),
        grid=(npad // BM,),
        in_specs=in_specs,
        out_specs=pl.BlockSpec((NSLAB, BM, SLAB), lambda m: (0, m, 0)),
        out_shape=jax.ShapeDtypeStruct((NSLAB, npad, SLAB), F32),
    )(x, wstk, *([s_hbm] * nrel), *([cnt2] * nrel))


def _logits_body(x_ref, hd_ref, xq_ref, o_ref):
    q = jnp.concatenate([xq_ref[k][0:1, :] for k in range(NSLAB)], axis=1)
    v = jnp.dot(hd_ref[...], q.reshape(HID, 1),
                preferred_element_type=F32)
    h = jnp.concatenate([x_ref[k] for k in range(NSLAB)], axis=1)
    o_ref[0, 0, :] = jnp.dot(h, v, preferred_element_type=F32)[:, 0]


def _logits(x, heads, xq, npad):
    nb = npad // BM
    out = pl.pallas_call(
        _logits_body,
        grid=(nb,),
        in_specs=[
            pl.BlockSpec((NSLAB, BM, SLAB), lambda m: (0, m, 0)),
            pl.BlockSpec((HID, HID), lambda m: (0, 0)),
            pl.BlockSpec((NSLAB, 8, SLAB), lambda m: (0, 0, 0)),
        ],
        out_specs=pl.BlockSpec((1, 1, BM), lambda m: (m, 0, 0)),
        out_shape=jax.ShapeDtypeStruct((nb, 1, BM), F32),
    )(x, heads, xq)
    return out.reshape(npad)


def _zero_stripes(dst_sh, zb, nrows, row0):
    ZR = zb.shape[0]
    nfull = nrows // ZR
    rem = nrows - nfull * ZR
    for i in range(nfull):
        pltpu.sync_copy(zb, dst_sh.at[pl.ds(row0 + i * ZR, ZR)])
    if rem:
        pltpu.sync_copy(zb.at[pl.ds(0, rem)],
                        dst_sh.at[pl.ds(row0 + nfull * ZR, rem)])


def _make_segsum(shapes):
    mesh = plsc.VectorSubcoreMesh(core_axis_name="core",
                                  subcore_axis_name="subcore",
                                  num_cores=NC, num_subcores=NS)
    tot = shapes['tot_slots']
    ZR = 8

    def body(*refs):
        nx = len(_NODE)
        x_refs = refs[:nx]
        src_hbm, dst_hbm, s_hbm = refs[nx], refs[nx + 1], refs[nx + 2]
        acc_sh, zb, src_i, dst_i, rows = refs[nx + 3:nx + 8]
        c = lax.axis_index("core")
        s = lax.axis_index("subcore")

        @pl.loop(0, ZR)
        def _(i):
            for kk in range(SLAB // 16):
                zb[i, pl.ds(kk * 16, 16)] = jnp.zeros((16,), F32)

        for ti, tname in enumerate(_NODE):
            for (src_blo, nb, dst_blo, accrows, crows, srow) in \
                    shapes['passes_by_src'][tname]:
                rp = accrows // NS
                _zero_stripes(acc_sh, zb, rp, s * rp)
                plsc.subcore_barrier()

                nt = (nb - s + NS - 1) // NS

                @pl.loop(0, nt)
                def _(t, src_blo=src_blo, dst_blo=dst_blo, xr=x_refs[ti]):
                    pltpu.sync_copy(src_hbm.at[src_blo + s + NS * t], src_i)
                    pltpu.sync_copy(dst_hbm.at[dst_blo + s + NS * t], dst_i)
                    pltpu.sync_copy(xr.at[c].at[src_i], rows)
                    pltpu.sync_copy(rows, acc_sh.at[dst_i], add=True)

                plsc.subcore_barrier()
                fp = crows // NS
                pltpu.sync_copy(acc_sh.at[pl.ds(s * fp, fp)],
                                s_hbm.at[c].at[pl.ds(srow + s * fp, fp)])
                plsc.subcore_barrier()

    @functools.partial(
        pl.kernel,
        out_type=jax.ShapeDtypeStruct((NSLAB, tot, SLAB), F32),
        mesh=mesh,
        scratch_types=[
            pltpu.VMEM_SHARED((MAXACC, SLAB), F32),
            pltpu.VMEM((ZR, SLAB), F32),
            pltpu.VMEM((BLK,), jnp.int32),
            pltpu.VMEM((BLK,), jnp.int32),
            pltpu.VMEM((BLK, SLAB), F32),
        ],
    )
    def k(*refs):
        body(*refs)

    return k


def _make_counts(shapes):
    mesh = plsc.VectorSubcoreMesh(core_axis_name="core",
                                  subcore_axis_name="subcore",
                                  num_cores=NC, num_subcores=NS)
    tot = shapes['tot_slots']
    sz0, sz1 = shapes['cnt_sz']
    b_split = shapes['cnt_bsplit']
    nbt = shapes['nb_tot']
    ZR = 8

    def core_pass(cid, dstg_hbm, cnt2_hbm, acc_sh, zb, idx_i, ones_v, s):
        sz = sz0 if cid == 0 else sz1
        blo = 0 if cid == 0 else b_split
        bhi = b_split if cid == 0 else nbt
        base = 0 if cid == 0 else sz0
        rp = sz // NS
        _zero_stripes(acc_sh, zb, rp, s * rp)
        plsc.subcore_barrier()
        nb = bhi - blo
        nt = (nb - s + NS - 1) // NS

        @pl.loop(0, nt)
        def _(t):
            pltpu.sync_copy(dstg_hbm.at[blo + s + NS * t], idx_i)
            pltpu.sync_copy(ones_v, acc_sh.at[idx_i], add=True)

        plsc.subcore_barrier()
        pltpu.sync_copy(acc_sh.at[pl.ds(s * rp, rp)],
                        cnt2_hbm.at[pl.ds(base + s * rp, rp)])

    @functools.partial(
        pl.kernel,
        out_type=jax.ShapeDtypeStruct((tot, 16), F32),
        mesh=mesh,
        scratch_types=[
            pltpu.VMEM_SHARED((max(sz0, sz1), 16), F32),
            pltpu.VMEM((ZR, 16), F32),
            pltpu.VMEM((BLK,), jnp.int32),
            pltpu.VMEM((BLK, 16), F32),
        ],
        compiler_params=pltpu.CompilerParams(use_tc_tiling_on_sc=False),
    )
    def k(dstg_hbm, cnt2_hbm, acc_sh, zb, idx_i, ones_v):
        c = lax.axis_index("core")
        s = lax.axis_index("subcore")

        @pl.loop(0, ZR)
        def _(i):
            zb[i, pl.ds(0, 16)] = jnp.zeros((16,), F32)

        @pl.loop(0, BLK)
        def _(i):
            ones_v[i, pl.ds(0, 16)] = jnp.ones((16,), F32)

        @pl.when(c == 0)
        def _():
            core_pass(0, dstg_hbm, cnt2_hbm, acc_sh, zb, idx_i, ones_v, s)

        @pl.when(c == 1)
        def _():
            core_pass(1, dstg_hbm, cnt2_hbm, acc_sh, zb, idx_i, ones_v, s)

    return k


def kernel(x_query, x_textblock, x_table, x_cell, x_image, x_caption,
           ei_query_to_text, ei_query_to_cell, ei_text_to_query,
           ei_cell_to_query, ei_table_contains_cell, ei_cell_in_table,
           ei_cell_to_cell_row, ei_cell_to_cell_col, ei_text_refers_table,
           ei_image_has_caption, ei_caption_to_image, params):
    xs = {'query': x_query, 'textblock': x_textblock, 'table': x_table,
          'cell': x_cell, 'image': x_image, 'caption': x_caption}
    eis = {'query_to_text': ei_query_to_text, 'query_to_cell': ei_query_to_cell,
           'text_to_query': ei_text_to_query, 'cell_to_query': ei_cell_to_query,
           'table_contains_cell': ei_table_contains_cell,
           'cell_in_table': ei_cell_in_table,
           'cell_to_cell_row': ei_cell_to_cell_row,
           'cell_to_cell_col': ei_cell_to_cell_col,
           'text_refers_table': ei_text_refers_table,
           'image_has_caption': ei_image_has_caption,
           'caption_to_image': ei_caption_to_image}

    nreal = {t: xs[t].shape[0] for t in _NODE}
    npad = {t: _rup(max(nreal[t], BM), BM) for t in _NODE}

    soff, ndp_of, cur = [], [], 0
    for (s_t, r, d_t) in _EDGES:
        soff.append(cur)
        ndp_of.append(npad[d_t])
        cur += npad[d_t]
    tot_slots = cur
    sz0 = sum(ndp_of[:_CNT_SPLIT])
    sz1 = tot_slots - sz0

    nb_r, blo_r, cur = [], [], 0
    for (s_t, r, d_t) in _EDGES:
        nb = _rup(eis[r].shape[1], BLK) // BLK
        blo_r.append(cur)
        nb_r.append(nb)
        cur += nb
    nb_tot = cur
    b_split = blo_r[_CNT_SPLIT]

    nchunk_r, crows_r = [], []
    for i in range(len(_EDGES)):
        ndp = ndp_of[i]
        nch = 1
        while True:
            crows = _rup(-(-ndp // nch), BM)
            if crows + (DUMMY if nch > 1 else 0) <= MAXACC:
                break
            nch += 1
        nchunk_r.append(nch)
        crows_r.append(crows)

    src_rows, dst_rows, dstg_rows = [], [], []
    dst_blo = {}
    nb2 = 0
    for i, (s_t, r, d_t) in enumerate(_EDGES):
        ei = eis[r]
        e = ei.shape[1]
        epad = nb_r[i] * BLK
        src = jnp.pad(ei[0].astype(jnp.int32), (0, epad - e))
        dst = jnp.pad(ei[1].astype(jnp.int32), (0, epad - e),
                      constant_values=nreal[d_t])
        src_rows.append(src.reshape(nb_r[i], BLK))
        base = 0 if i < _CNT_SPLIT else sz0
        dstg_rows.append((dst + (soff[i] - base)).reshape(nb_r[i], BLK))
        crows = crows_r[i]
        spread = jnp.arange(epad, dtype=jnp.int32) % DUMMY
        for ch in range(nchunk_r[i]):
            lo = ch * crows
            hi = min(lo + crows, ndp_of[i])
            inrange = jnp.logical_and(dst >= lo, dst < hi)
            dloc = jnp.where(inrange, dst - lo, crows + spread)
            dst_rows.append(dloc.reshape(nb_r[i], BLK))
            dst_blo[(i, ch)] = nb2
            nb2 += nb_r[i]
    SRC = jnp.concatenate(src_rows, 0)
    DST = jnp.concatenate(dst_rows, 0)
    DSTG = jnp.concatenate(dstg_rows, 0)

    passes_by_src = {t: [] for t in _NODE}
    for i, (s_t, r, d_t) in enumerate(_EDGES):
        for ch in range(nchunk_r[i]):
            crows = crows_r[i]
            accrows = crows + (DUMMY if nchunk_r[i] > 1 else 0)
            accrows = _rup(accrows, NS * 16)
            frows = min(crows, ndp_of[i] - ch * crows)
            passes_by_src[s_t].append(
                (blo_r[i], nb_r[i], dst_blo[(i, ch)], accrows, frows,
                 soff[i] + ch * crows))

    shapes = {
        'tot_slots': tot_slots,
        'nb_tot': nb_tot,
        'cnt_sz': (sz0, sz1),
        'cnt_bsplit': b_split,
        'passes_by_src': passes_by_src,
    }

    x = {}
    for t in _NODE:
        xp = jnp.pad(xs[t], ((0, npad[t] - nreal[t]), (0, 0)))
        x[t] = _proj(xp, params['proj'][t]['W'], params['proj'][t]['b'],
                     npad[t])

    cnt2 = _make_counts(shapes)(DSTG)

    segsum = _make_segsum(shapes)
    for lp in params['layers']:
        s_hbm = segsum(*[x[t] for t in _NODE], SRC, DST)
        xn = {}
        for t in _NODE:
            rels = _RELS_OF[t]
            wstk = jnp.concatenate(
                [lp['self'][t]] + [lp['rel'][_EDGES[i][1]] for i in rels], 0)
            xn[t] = _layer_mm(x[t], s_hbm, cnt2, wstk, npad[t],
                              [soff[i] for i in rels])
        x = xn

    out = []
    for t in _SCORED:
        lg = _logits(x[t], params['heads'][t], x['query'], npad[t])
        out.append(lg[:nreal[t]])
    return tuple(out)
